# baseline (speedup 1.0000x reference)
import functools

import jax
import jax.numpy as jnp
from jax import lax
from jax.experimental import pallas as pl
from jax.experimental.layout import Format, Layout
from jax.experimental.pallas import tpu as pltpu
from jax.experimental.pallas import tpu_sc as plsc

NC, NS, L = 2, 16, 16
NW = NC * NS
MAXL = 200
NN = 16384
TE = 1638400
WPN = 3 * MAXL
NPW = NN // NW
SPS = NN // NS
NV = SPS // L

_mesh = plsc.VectorSubcoreMesh(
    core_axis_name="c", subcore_axis_name="s", num_cores=NC, num_subcores=NS
)
_params = pltpu.CompilerParams(needs_layout_passes=False)

_I31 = 0x7FFFFFFF


def _lane():
    return lax.iota(jnp.int32, L)


@functools.partial(
    pl.kernel,
    out_type=jax.ShapeDtypeStruct((2 * NN,), jnp.int32),
    mesh=_mesh,
    compiler_params=_params,
    scratch_types=[
        pltpu.VMEM((SPS,), jnp.float32),
        pltpu.VMEM((SPS,), jnp.int32),
        pltpu.VMEM((SPS,), jnp.int32),
        pltpu.VMEM((SPS,), jnp.int32),
        pltpu.VMEM((SPS,), jnp.int32),
        pltpu.VMEM((SPS,), jnp.int32),
        pltpu.VMEM((SPS,), jnp.int32),
        pltpu.VMEM_SHARED((NN,), jnp.int32),
        pltpu.VMEM_SHARED((NN,), jnp.int32),
    ],
)
def _sort_kernel(comb_hbm, out_hbm,
                 fb, ka, va, kb, vb, pk, pv, shk, shv):
    c = lax.axis_index("c")
    s = lax.axis_index("s")
    base = s * SPS
    cbase = c * NN + base
    lane = _lane()

    pltpu.sync_copy(comb_hbm.at[pl.ds(cbase, SPS)], fb)

    for t in range(NV):
        i = plsc.bitcast(fb[pl.ds(t * L, L)], jnp.int32)
        ka[pl.ds(t * L, L)] = jnp.where(i >= 0, i, i ^ _I31)
        va[pl.ds(t * L, L)] = lane + (base + t * L)

    def do_pass(kk, jj, src_k, src_v, dst_k, dst_v):
        @pl.when(jj >= SPS)
        def _():
            pltpu.sync_copy(src_k, shk.at[pl.ds(base, SPS)])
            pltpu.sync_copy(src_v, shv.at[pl.ds(base, SPS)])
            plsc.subcore_barrier()
            ps = s ^ (jj // SPS)
            pltpu.sync_copy(shk.at[pl.ds(ps * SPS, SPS)], pk)
            pltpu.sync_copy(shv.at[pl.ds(ps * SPS, SPS)], pv)
            is_lo = (s & (jj // SPS)) == 0
            asc = (base & kk) == 0
            take_min = is_lo == asc

            def t_body(t, _):
                sl = pl.ds(t * L, L)
                xk, xv = src_k[sl], src_v[sl]
                yk, yv = pk[sl], pv[sl]
                ltv = (xk < yk) | ((xk == yk) & (xv < yv))
                cond = ltv == take_min
                dst_k[sl] = jnp.where(cond, xk, yk)
                dst_v[sl] = jnp.where(cond, xv, yv)
                return 0

            lax.fori_loop(0, NV, t_body, 0)
            plsc.subcore_barrier()

        @pl.when((jj >= L) & (jj < SPS))
        def _():
            jv = jj // L

            def q_body(q, _):
                lo = q & (jv - 1)
                t = (q - lo) * 2 + lo
                t2 = t + jv
                s1 = pl.ds(t * L, L)
                s2 = pl.ds(t2 * L, L)
                asc = ((base + t * L) & kk) == 0
                xk, xv = src_k[s1], src_v[s1]
                yk, yv = src_k[s2], src_v[s2]
                ltv = (xk < yk) | ((xk == yk) & (xv < yv))
                cond = ltv == asc
                dst_k[s1] = jnp.where(cond, xk, yk)
                dst_v[s1] = jnp.where(cond, xv, yv)
                dst_k[s2] = jnp.where(cond, yk, xk)
                dst_v[s2] = jnp.where(cond, yv, xv)
                return 0

            lax.fori_loop(0, NV // 2, q_body, 0)

        @pl.when(jj < L)
        def _():
            perm = lane ^ jj
            is_lo = (lane & jj) == 0

            def t_body(t, _):
                sl = pl.ds(t * L, L)
                xk, xv = src_k[sl], src_v[sl]
                gi = perm + t * L
                yk = plsc.load_gather(src_k, [gi])
                yv = plsc.load_gather(src_v, [gi])
                asc = (((base + t * L) + lane) & kk) == 0
                take_min = is_lo == asc
                ltv = (xk < yk) | ((xk == yk) & (xv < yv))
                cond = ltv == take_min
                dst_k[sl] = jnp.where(cond, xk, yk)
                dst_v[sl] = jnp.where(cond, xv, yv)
                return 0

            lax.fori_loop(0, NV, t_body, 0)

    def stage_body(st_i, ph):
        kk = lax.shift_left(jnp.int32(1), st_i)

        def j_body(m, ph):
            jj = lax.shift_left(jnp.int32(1), st_i - 1 - m)

            @pl.when(ph == 0)
            def _():
                do_pass(kk, jj, ka, va, kb, vb)

            @pl.when(ph == 1)
            def _():
                do_pass(kk, jj, kb, vb, ka, va)

            return ph ^ 1

        return lax.fori_loop(0, st_i, j_body, ph)

    lax.fori_loop(1, 15, stage_body, jnp.int32(0))

    for t in range(NV):
        sl = pl.ds(t * L, L)
        k = kb[sl]
        bits = jnp.where(k >= 0, k, k ^ _I31)
        pk[sl] = jnp.where(c == 0, vb[sl], bits)
    pltpu.sync_copy(pk, out_hbm.at[pl.ds(cbase, SPS)])


@functools.partial(
    pl.kernel,
    out_type=jax.ShapeDtypeStruct((NN,), jnp.int32),
    mesh=_mesh,
    compiler_params=_params,
    scratch_types=[
        pltpu.VMEM((NN,), jnp.float32),
        pltpu.VMEM((NPW // 128, 128), jnp.int32),
        pltpu.VMEM((NPW,), jnp.int32),
        pltpu.SemaphoreType.DMA,
    ],
)
def _final_kernel(gidx_hbm, st_hbm, map_hbm, st_v, gi_v, val_v, sem):
    c = lax.axis_index("c")
    s = lax.axis_index("s")
    wid = s * NC + c
    base = wid * NPW
    lane = _lane()

    pltpu.sync_copy(st_hbm, st_v)
    pltpu.sync_copy(gidx_hbm.at[pl.ds(wid * (NPW // 128), NPW // 128)], gi_v)

    nm1 = jnp.float32(NN - 1)
    for t in range(NPW // L):
        r = (base + t * L) + lane
        ii = (r.astype(jnp.float32) / nm1) * nm1
        fl = ii.astype(jnp.int32)
        flf = fl.astype(jnp.float32)
        ce = jnp.minimum(fl + (ii > flf).astype(jnp.int32), NN - 1)
        wc = ii - flf
        a = plsc.load_gather(st_v, [fl])
        b = plsc.load_gather(st_v, [ce])
        val = (jnp.float32(1.0) - wc) * a + wc * b
        val_v[pl.ds(t * L, L)] = val.astype(jnp.int32)

    for q in range(NPW // 128):
        pltpu.async_copy(
            val_v.at[pl.ds(q * 128, 128)], map_hbm.at[gi_v.at[q]], sem
        ).wait()


@functools.partial(
    pl.kernel,
    out_type=(
        jax.ShapeDtypeStruct((NN * WPN,), jnp.float32),
        jax.ShapeDtypeStruct((NN * MAXL,), jnp.int32),
        jax.ShapeDtypeStruct((NN,), jnp.int32),
    ),
    mesh=_mesh,
    compiler_params=_params,
    scratch_types=[
        pltpu.VMEM((NPW + 16,), jnp.int32),
        pltpu.VMEM((1216,), jnp.float32),
        pltpu.VMEM((640,), jnp.float32),
        pltpu.VMEM((208,), jnp.int32),
        pltpu.VMEM((NPW,), jnp.int32),
    ],
)
def _pad_kernel(ef_hbm, cu_hbm, pad_hbm, int_hbm, len_hbm,
                cu_v, in_v, out_v, int_v, len_v):
    c = lax.axis_index("c")
    s = lax.axis_index("s")
    wid = s * NC + c
    base = wid * NPW
    lane = _lane()

    pltpu.sync_copy(cu_hbm.at[pl.ds(base, NPW + 16)], cu_v)

    def body(n, _):
        cuv = cu_v[pl.ds(n, L)]
        start = cuv[0]
        end = cuv[1]
        ln = end - start
        len_c = jnp.minimum(ln, MAXL)
        add1 = jnp.where(ln <= MAXL, jnp.float32(1.0), jnp.float32(0.0))
        w0 = start * 3
        a0 = jnp.minimum(w0 - (w0 & 7), 3 * TE - 608)
        a0 = pl.multiple_of(a0, 8)
        d = w0 - a0
        pltpu.sync_copy(ef_hbm.at[pl.ds(a0, 608)], in_v.at[pl.ds(0, 608)])
        for t in range(38):
            p = (lane + t * L) // 3
            x = in_v[pl.ds(d + t * L, L)]
            out_v[pl.ds(t * L, L)] = jnp.where(p < len_c, x + add1,
                                               jnp.float32(0.0))
        for t in range(13):
            gi = lane * 3 + (2 + t * 3 * L)
            v = plsc.load_gather(out_v, [gi])
            int_v[pl.ds(t * L, L)] = v.astype(jnp.int32)
        g = base + n
        pltpu.sync_copy(out_v.at[pl.ds(0, WPN)],
                        pad_hbm.at[pl.ds(g * WPN, WPN)])
        pltpu.sync_copy(int_v.at[pl.ds(0, MAXL)],
                        int_hbm.at[pl.ds(g * MAXL, MAXL)])
        return 0

    lax.fori_loop(0, NPW, body, 0)
    for t in range(NPW // L):
        starts = cu_v[pl.ds(t * L, L)]
        ends = cu_v[pl.ds(t * L + 1, L)]
        len_v[pl.ds(t * L, L)] = jnp.minimum(ends - starts, MAXL)
    pltpu.sync_copy(len_v, len_hbm.at[pl.ds(base, NPW)])


_sharding = jax.sharding.SingleDeviceSharding(jax.devices()[0])
_out_formats = (
    Format(Layout((0, 1, 2), ()), _sharding),
    Format(Layout((0,), ()), _sharding),
    Format(Layout((0, 1), ()), _sharding),
    Format(Layout((0, 1), ()), _sharding),
)


@functools.partial(jax.jit, out_shardings=_out_formats)
def kernel(edges_flat, cu_seqlens, generated_data, target_quantile):
    ef = edges_flat.reshape(-1)
    cu_pad = jnp.concatenate(
        [cu_seqlens, jnp.full((31,), TE, jnp.int32)])
    gen = generated_data.reshape(-1)

    comb = jnp.concatenate([gen, target_quantile])
    sorted_comb = _sort_kernel(comb)
    gidx = sorted_comb[:NN]
    st = lax.bitcast_convert_type(sorted_comb[NN:], jnp.float32)
    mapped = _final_kernel(gidx.reshape(NN // 128, 128), st)
    padded_flat, int_flat, lens = _pad_kernel(ef, cu_pad)

    return (padded_flat.reshape(NN, MAXL, 3), lens,
            int_flat.reshape(NN, MAXL), mapped.reshape(NN, 1))


# speedup vs baseline: 1.5103x; 1.5103x over previous
import functools

import jax
import jax.numpy as jnp
from jax import lax
from jax.experimental import pallas as pl
from jax.experimental.layout import Format, Layout
from jax.experimental.pallas import tpu as pltpu
from jax.experimental.pallas import tpu_sc as plsc

NC, NS, L = 2, 16, 16
NW = NC * NS
MAXL = 200
NN = 16384
TE = 1638400
WPN = 3 * MAXL
NPW = NN // NW
SPS = NN // NS
NV = SPS // L

_mesh = plsc.VectorSubcoreMesh(
    core_axis_name="c", subcore_axis_name="s", num_cores=NC, num_subcores=NS
)
_params = pltpu.CompilerParams(needs_layout_passes=False,
                               use_tc_tiling_on_sc=False)

_I31 = 0x7FFFFFFF


def _lane():
    return lax.iota(jnp.int32, L)


@functools.partial(
    pl.kernel,
    out_type=jax.ShapeDtypeStruct((2 * NN,), jnp.int32),
    mesh=_mesh,
    compiler_params=_params,
    scratch_types=[
        pltpu.VMEM((SPS,), jnp.float32),
        pltpu.VMEM((SPS,), jnp.int32),
        pltpu.VMEM((SPS,), jnp.int32),
        pltpu.VMEM((SPS,), jnp.int32),
        pltpu.VMEM((SPS,), jnp.int32),
        pltpu.VMEM((SPS,), jnp.int32),
        pltpu.VMEM((SPS,), jnp.int32),
        pltpu.VMEM_SHARED((NN,), jnp.int32),
        pltpu.VMEM_SHARED((NN,), jnp.int32),
    ],
)
def _sort_kernel(comb_hbm, out_hbm,
                 fb, ka, va, kb, vb, pk, pv, shk, shv):
    c = lax.axis_index("c")
    s = lax.axis_index("s")
    base = s * SPS
    cbase = c * NN + base
    lane = _lane()

    pltpu.sync_copy(comb_hbm.at[pl.ds(cbase, SPS)], fb)

    for t in range(NV):
        i = plsc.bitcast(fb[pl.ds(t * L, L)], jnp.int32)
        ka[pl.ds(t * L, L)] = jnp.where(i >= 0, i, i ^ _I31)
        va[pl.ds(t * L, L)] = lane + (base + t * L)

    def do_pass(kk, jj, src_k, src_v, dst_k, dst_v):
        @pl.when(jj >= SPS)
        def _():
            pltpu.sync_copy(src_k, shk.at[pl.ds(base, SPS)])
            pltpu.sync_copy(src_v, shv.at[pl.ds(base, SPS)])
            plsc.subcore_barrier()
            ps = s ^ (jj // SPS)
            pltpu.sync_copy(shk.at[pl.ds(ps * SPS, SPS)], pk)
            pltpu.sync_copy(shv.at[pl.ds(ps * SPS, SPS)], pv)
            is_lo = (s & (jj // SPS)) == 0
            asc = (base & kk) == 0
            take_min = is_lo == asc

            def t_body(t, _):
                sl = pl.ds(t * L, L)
                xk, xv = src_k[sl], src_v[sl]
                yk, yv = pk[sl], pv[sl]
                ltv = (xk < yk) | ((xk == yk) & (xv < yv))
                cond = ltv == take_min
                dst_k[sl] = jnp.where(cond, xk, yk)
                dst_v[sl] = jnp.where(cond, xv, yv)
                return 0

            lax.fori_loop(0, NV, t_body, 0)
            plsc.subcore_barrier()

        @pl.when((jj >= L) & (jj < SPS))
        def _():
            jv = jj // L

            def q_body(q, _):
                lo = q & (jv - 1)
                t = (q - lo) * 2 + lo
                t2 = t + jv
                s1 = pl.ds(t * L, L)
                s2 = pl.ds(t2 * L, L)
                asc = ((base + t * L) & kk) == 0
                xk, xv = src_k[s1], src_v[s1]
                yk, yv = src_k[s2], src_v[s2]
                ltv = (xk < yk) | ((xk == yk) & (xv < yv))
                cond = ltv == asc
                dst_k[s1] = jnp.where(cond, xk, yk)
                dst_v[s1] = jnp.where(cond, xv, yv)
                dst_k[s2] = jnp.where(cond, yk, xk)
                dst_v[s2] = jnp.where(cond, yv, xv)
                return 0

            lax.fori_loop(0, NV // 2, q_body, 0)

        @pl.when(jj < L)
        def _():
            perm = lane ^ jj
            is_lo = (lane & jj) == 0

            def t_body(t, _):
                sl = pl.ds(t * L, L)
                xk, xv = src_k[sl], src_v[sl]
                gi = perm + t * L
                yk = plsc.load_gather(src_k, [gi])
                yv = plsc.load_gather(src_v, [gi])
                asc = (((base + t * L) + lane) & kk) == 0
                take_min = is_lo == asc
                ltv = (xk < yk) | ((xk == yk) & (xv < yv))
                cond = ltv == take_min
                dst_k[sl] = jnp.where(cond, xk, yk)
                dst_v[sl] = jnp.where(cond, xv, yv)
                return 0

            lax.fori_loop(0, NV, t_body, 0)

    def stage_body(st_i, ph):
        kk = lax.shift_left(jnp.int32(1), st_i)

        def j_body(m, ph):
            jj = lax.shift_left(jnp.int32(1), st_i - 1 - m)

            @pl.when(ph == 0)
            def _():
                do_pass(kk, jj, ka, va, kb, vb)

            @pl.when(ph == 1)
            def _():
                do_pass(kk, jj, kb, vb, ka, va)

            return ph ^ 1

        return lax.fori_loop(0, st_i, j_body, ph)

    lax.fori_loop(1, 15, stage_body, jnp.int32(0))

    for t in range(NV):
        sl = pl.ds(t * L, L)
        k = kb[sl]
        bits = jnp.where(k >= 0, k, k ^ _I31)
        pk[sl] = jnp.where(c == 0, vb[sl], bits)
    pltpu.sync_copy(pk, out_hbm.at[pl.ds(cbase, SPS)])


@functools.partial(
    pl.kernel,
    out_type=jax.ShapeDtypeStruct((NN,), jnp.int32),
    mesh=_mesh,
    compiler_params=_params,
    scratch_types=[
        pltpu.VMEM((NN,), jnp.float32),
        pltpu.VMEM((NPW // 128, 128), jnp.int32),
        pltpu.VMEM((NPW,), jnp.int32),
        pltpu.SemaphoreType.DMA,
    ],
)
def _final_kernel(gidx_hbm, st_hbm, map_hbm, st_v, gi_v, val_v, sem):
    c = lax.axis_index("c")
    s = lax.axis_index("s")
    wid = s * NC + c
    base = wid * NPW
    lane = _lane()

    pltpu.sync_copy(st_hbm, st_v)
    pltpu.sync_copy(gidx_hbm.at[pl.ds(wid * (NPW // 128), NPW // 128)], gi_v)

    nm1 = jnp.float32(NN - 1)
    for t in range(NPW // L):
        r = (base + t * L) + lane
        ii = (r.astype(jnp.float32) / nm1) * nm1
        fl = ii.astype(jnp.int32)
        flf = fl.astype(jnp.float32)
        ce = jnp.minimum(fl + (ii > flf).astype(jnp.int32), NN - 1)
        wc = ii - flf
        a = plsc.load_gather(st_v, [fl])
        b = plsc.load_gather(st_v, [ce])
        val = (jnp.float32(1.0) - wc) * a + wc * b
        val_v[pl.ds(t * L, L)] = val.astype(jnp.int32)

    for q in range(NPW // 128):
        pltpu.async_copy(
            val_v.at[pl.ds(q * 128, 128)], map_hbm.at[gi_v.at[q]], sem
        ).wait()


@functools.partial(
    pl.kernel,
    out_type=(
        jax.ShapeDtypeStruct((NN * WPN,), jnp.float32),
        jax.ShapeDtypeStruct((NN * MAXL,), jnp.int32),
        jax.ShapeDtypeStruct((NN,), jnp.int32),
    ),
    mesh=_mesh,
    compiler_params=_params,
    scratch_types=[
        pltpu.VMEM((NPW + 16,), jnp.int32),
        pltpu.VMEM((1536,), jnp.float32),
        pltpu.VMEM((640,), jnp.float32),
        pltpu.VMEM((208,), jnp.int32),
        pltpu.VMEM((NPW,), jnp.int32),
    ],
)
def _pad_kernel(e0_hbm, e1_hbm, e2_hbm, cu_hbm, pad_hbm, int_hbm, len_hbm,
                cu_v, in_v, out_v, int_v, len_v):
    c = lax.axis_index("c")
    s = lax.axis_index("s")
    wid = s * NC + c
    base = wid * NPW
    lane = _lane()

    pltpu.sync_copy(cu_hbm.at[pl.ds(base, NPW + 16)], cu_v)

    def body(n, _):
        cuv = cu_v[pl.ds(n, L)]
        start = cuv[0]
        end = cuv[1]
        ln = end - start
        len_c = jnp.minimum(ln, MAXL)
        add1 = jnp.where(ln <= MAXL, jnp.float32(1.0), jnp.float32(0.0))
        a0 = jnp.minimum(start - (start & 7), TE - 208)
        a0 = pl.multiple_of(a0, 8)
        d = start - a0
        pltpu.sync_copy(e0_hbm.at[pl.ds(a0, 208)], in_v.at[pl.ds(0, 208)])
        pltpu.sync_copy(e1_hbm.at[pl.ds(a0, 208)], in_v.at[pl.ds(512, 208)])
        pltpu.sync_copy(e2_hbm.at[pl.ds(a0, 208)], in_v.at[pl.ds(1024, 208)])
        for t in range(38):
            w = lane + t * L
            p = w // 3
            gi = (w % 3) * 512 + p + d
            x = plsc.load_gather(in_v, [gi])
            out_v[pl.ds(t * L, L)] = jnp.where(p < len_c, x + add1,
                                               jnp.float32(0.0))
        for t in range(13):
            p = lane + t * L
            x2 = in_v[pl.ds(1024 + d + t * L, L)]
            y = jnp.where(p < len_c, x2 + add1, jnp.float32(0.0))
            int_v[pl.ds(t * L, L)] = y.astype(jnp.int32)
        g = base + n
        pltpu.sync_copy(out_v.at[pl.ds(0, WPN)],
                        pad_hbm.at[pl.ds(g * WPN, WPN)])
        pltpu.sync_copy(int_v.at[pl.ds(0, MAXL)],
                        int_hbm.at[pl.ds(g * MAXL, MAXL)])
        return 0

    lax.fori_loop(0, NPW, body, 0)
    for t in range(NPW // L):
        starts = cu_v[pl.ds(t * L, L)]
        ends = cu_v[pl.ds(t * L + 1, L)]
        len_v[pl.ds(t * L, L)] = jnp.minimum(ends - starts, MAXL)
    pltpu.sync_copy(len_v, len_hbm.at[pl.ds(base, NPW)])


_sharding = jax.sharding.SingleDeviceSharding(jax.devices()[0])
_out_formats = (
    Format(Layout((0, 1, 2), ((8, 3),)), _sharding),
    Format(Layout((0,), None), _sharding),
    Format(Layout((0, 1), ((8, 200),)), _sharding),
    Format(Layout((0, 1), ((8, 1),)), _sharding),
)


@jax.jit
def kernel(edges_flat, cu_seqlens, generated_data, target_quantile):
    e0 = edges_flat[:, 0]
    e1 = edges_flat[:, 1]
    e2 = edges_flat[:, 2]
    cu_pad = jnp.concatenate(
        [cu_seqlens, jnp.full((31,), TE, jnp.int32)])
    gen = generated_data.reshape(-1)

    comb = jnp.concatenate([gen, target_quantile])
    sorted_comb = _sort_kernel(comb)
    gidx = sorted_comb[:NN]
    st = lax.bitcast_convert_type(sorted_comb[NN:], jnp.float32)
    mapped = _final_kernel(gidx.reshape(NN // 128, 128), st)
    padded_flat, int_flat, lens = _pad_kernel(e0, e1, e2, cu_pad)

    return (padded_flat.reshape(NN, MAXL, 3), lens,
            int_flat.reshape(NN, MAXL), mapped.reshape(NN, 1))


# speedup vs baseline: 1.9216x; 1.2723x over previous
import functools

import jax
import jax.numpy as jnp
from jax import lax
from jax.experimental import pallas as pl
from jax.experimental.layout import Format, Layout
from jax.experimental.pallas import tpu as pltpu
from jax.experimental.pallas import tpu_sc as plsc

NC, NS, L = 2, 16, 16
NW = NC * NS
MAXL = 200
NN = 16384
TE = 1638400
WPN = 3 * MAXL
NPW = NN // NW
SPS = NN // NS
NV = SPS // L

_mesh = plsc.VectorSubcoreMesh(
    core_axis_name="c", subcore_axis_name="s", num_cores=NC, num_subcores=NS
)
_params = pltpu.CompilerParams(needs_layout_passes=False,
                               use_tc_tiling_on_sc=False)

_I31 = 0x7FFFFFFF


def _lane():
    return lax.iota(jnp.int32, L)


@functools.partial(
    pl.kernel,
    out_type=jax.ShapeDtypeStruct((2 * NN,), jnp.int32),
    mesh=_mesh,
    compiler_params=_params,
    scratch_types=[
        pltpu.VMEM((SPS,), jnp.float32),
        pltpu.VMEM((SPS,), jnp.int32),
        pltpu.VMEM((SPS,), jnp.int32),
        pltpu.VMEM((SPS,), jnp.int32),
        pltpu.VMEM((SPS,), jnp.int32),
        pltpu.VMEM((SPS,), jnp.int32),
        pltpu.VMEM((SPS,), jnp.int32),
        pltpu.VMEM_SHARED((NN,), jnp.int32),
        pltpu.VMEM_SHARED((NN,), jnp.int32),
    ],
)
def _sort_kernel(comb_hbm, out_hbm,
                 fb, ka, va, kb, vb, pk, pv, shk, shv):
    c = lax.axis_index("c")
    s = lax.axis_index("s")
    base = s * SPS
    cbase = c * NN + base
    lane = _lane()

    pltpu.sync_copy(comb_hbm.at[pl.ds(cbase, SPS)], fb)

    for t in range(NV):
        i = plsc.bitcast(fb[pl.ds(t * L, L)], jnp.int32)
        ka[pl.ds(t * L, L)] = jnp.where(i >= 0, i, i ^ _I31)
        va[pl.ds(t * L, L)] = lane + (base + t * L)

    def do_pass(kk, jj, src_k, src_v, dst_k, dst_v):
        @pl.when(jj >= SPS)
        def _():
            pltpu.sync_copy(src_k, shk.at[pl.ds(base, SPS)])
            pltpu.sync_copy(src_v, shv.at[pl.ds(base, SPS)])
            plsc.subcore_barrier()
            ps = s ^ (jj // SPS)
            pltpu.sync_copy(shk.at[pl.ds(ps * SPS, SPS)], pk)
            pltpu.sync_copy(shv.at[pl.ds(ps * SPS, SPS)], pv)
            is_lo = (s & (jj // SPS)) == 0
            asc = (base & kk) == 0
            take_min = is_lo == asc

            def t_body(t, _):
                sl = pl.ds(t * L, L)
                xk, xv = src_k[sl], src_v[sl]
                yk, yv = pk[sl], pv[sl]
                ltv = (xk < yk) | ((xk == yk) & (xv < yv))
                cond = ltv == take_min
                dst_k[sl] = jnp.where(cond, xk, yk)
                dst_v[sl] = jnp.where(cond, xv, yv)
                return 0

            lax.fori_loop(0, NV, t_body, 0)
            plsc.subcore_barrier()

        @pl.when((jj >= L) & (jj < SPS))
        def _():
            jv = jj // L

            def q_body(q, _):
                lo = q & (jv - 1)
                t = (q - lo) * 2 + lo
                t2 = t + jv
                s1 = pl.ds(t * L, L)
                s2 = pl.ds(t2 * L, L)
                asc = ((base + t * L) & kk) == 0
                xk, xv = src_k[s1], src_v[s1]
                yk, yv = src_k[s2], src_v[s2]
                ltv = (xk < yk) | ((xk == yk) & (xv < yv))
                cond = ltv == asc
                dst_k[s1] = jnp.where(cond, xk, yk)
                dst_v[s1] = jnp.where(cond, xv, yv)
                dst_k[s2] = jnp.where(cond, yk, xk)
                dst_v[s2] = jnp.where(cond, yv, xv)
                return 0

            lax.fori_loop(0, NV // 2, q_body, 0)

        @pl.when(jj < L)
        def _():
            perm = lane ^ jj
            is_lo = (lane & jj) == 0

            def t_body(t, _):
                sl = pl.ds(t * L, L)
                xk, xv = src_k[sl], src_v[sl]
                gi = perm + t * L
                yk = plsc.load_gather(src_k, [gi])
                yv = plsc.load_gather(src_v, [gi])
                asc = (((base + t * L) + lane) & kk) == 0
                take_min = is_lo == asc
                ltv = (xk < yk) | ((xk == yk) & (xv < yv))
                cond = ltv == take_min
                dst_k[sl] = jnp.where(cond, xk, yk)
                dst_v[sl] = jnp.where(cond, xv, yv)
                return 0

            lax.fori_loop(0, NV, t_body, 0)

    def stage_body(st_i, ph):
        kk = lax.shift_left(jnp.int32(1), st_i)

        def j_body(m, ph):
            jj = lax.shift_left(jnp.int32(1), st_i - 1 - m)

            @pl.when(ph == 0)
            def _():
                do_pass(kk, jj, ka, va, kb, vb)

            @pl.when(ph == 1)
            def _():
                do_pass(kk, jj, kb, vb, ka, va)

            return ph ^ 1

        return lax.fori_loop(0, st_i, j_body, ph)

    lax.fori_loop(1, 15, stage_body, jnp.int32(0))

    for t in range(NV):
        sl = pl.ds(t * L, L)
        k = kb[sl]
        bits = jnp.where(k >= 0, k, k ^ _I31)
        pk[sl] = jnp.where(c == 0, vb[sl], bits)
    pltpu.sync_copy(pk, out_hbm.at[pl.ds(cbase, SPS)])


@functools.partial(
    pl.kernel,
    out_type=jax.ShapeDtypeStruct((NN,), jnp.int32),
    mesh=_mesh,
    compiler_params=_params,
    scratch_types=[
        pltpu.VMEM((NN,), jnp.float32),
        pltpu.VMEM((NPW // 128, 128), jnp.int32),
        pltpu.VMEM((NPW,), jnp.int32),
        pltpu.SemaphoreType.DMA,
    ],
)
def _final_kernel(gidx_hbm, st_hbm, map_hbm, st_v, gi_v, val_v, sem):
    c = lax.axis_index("c")
    s = lax.axis_index("s")
    wid = s * NC + c
    base = wid * NPW
    lane = _lane()

    pltpu.sync_copy(st_hbm, st_v)
    pltpu.sync_copy(gidx_hbm.at[pl.ds(wid * (NPW // 128), NPW // 128)], gi_v)

    nm1 = jnp.float32(NN - 1)
    for t in range(NPW // L):
        r = (base + t * L) + lane
        ii = (r.astype(jnp.float32) / nm1) * nm1
        fl = ii.astype(jnp.int32)
        flf = fl.astype(jnp.float32)
        ce = jnp.minimum(fl + (ii > flf).astype(jnp.int32), NN - 1)
        wc = ii - flf
        a = plsc.load_gather(st_v, [fl])
        b = plsc.load_gather(st_v, [ce])
        val = (jnp.float32(1.0) - wc) * a + wc * b
        val_v[pl.ds(t * L, L)] = val.astype(jnp.int32)

    for q in range(NPW // 128):
        pltpu.async_copy(
            val_v.at[pl.ds(q * 128, 128)], map_hbm.at[gi_v.at[q]], sem
        ).wait()


@functools.partial(
    pl.kernel,
    out_type=(
        jax.ShapeDtypeStruct((NN * WPN,), jnp.float32),
        jax.ShapeDtypeStruct((NN * MAXL,), jnp.int32),
        jax.ShapeDtypeStruct((NN,), jnp.int32),
    ),
    mesh=_mesh,
    compiler_params=_params,
    scratch_types=[
        pltpu.VMEM((NPW + 16,), jnp.int32),
        pltpu.VMEM((3072,), jnp.float32),
        pltpu.VMEM((1280,), jnp.float32),
        pltpu.VMEM((512,), jnp.int32),
        pltpu.VMEM((NPW,), jnp.int32),
        pltpu.SemaphoreType.DMA,
        pltpu.SemaphoreType.DMA,
        pltpu.SemaphoreType.DMA,
        pltpu.SemaphoreType.DMA,
    ],
)
def _pad_kernel(e0_hbm, e1_hbm, e2_hbm, cu_hbm, pad_hbm, int_hbm, len_hbm,
                cu_v, in_v, out_v, int_v, len_v,
                sin0, sin1, sout0, sout1):
    c = lax.axis_index("c")
    s = lax.axis_index("s")
    wid = s * NC + c
    base = wid * NPW
    lane = _lane()

    pltpu.sync_copy(cu_hbm.at[pl.ds(base, NPW + 16)], cu_v)

    def node_params(n):
        cuv = cu_v[pl.ds(n, L)]
        start = cuv[0]
        end = cuv[1]
        ln = end - start
        a0 = jnp.minimum(start - (start & 7), TE - 208)
        a0 = pl.multiple_of(a0, 8)
        return start, ln, a0

    def issue_in(n, slot_off, sem):
        _, _, a0 = node_params(n)
        pltpu.async_copy(e0_hbm.at[pl.ds(a0, 208)],
                         in_v.at[pl.ds(slot_off, 208)], sem)
        pltpu.async_copy(e1_hbm.at[pl.ds(a0, 208)],
                         in_v.at[pl.ds(slot_off + 512, 208)], sem)
        pltpu.async_copy(e2_hbm.at[pl.ds(a0, 208)],
                         in_v.at[pl.ds(slot_off + 1024, 208)], sem)

    def wait_in(slot_off, sem):
        pltpu.make_async_copy(e0_hbm.at[pl.ds(0, 208)],
                              in_v.at[pl.ds(slot_off, 208)], sem).wait()
        pltpu.make_async_copy(e1_hbm.at[pl.ds(0, 208)],
                              in_v.at[pl.ds(slot_off + 512, 208)], sem).wait()
        pltpu.make_async_copy(e2_hbm.at[pl.ds(0, 208)],
                              in_v.at[pl.ds(slot_off + 1024, 208)], sem).wait()

    def wait_out(o_off, i_off, sem):
        pltpu.make_async_copy(out_v.at[pl.ds(o_off, WPN)],
                              pad_hbm.at[pl.ds(0, WPN)], sem).wait()
        pltpu.make_async_copy(int_v.at[pl.ds(i_off, MAXL)],
                              int_hbm.at[pl.ds(0, MAXL)], sem).wait()

    def compute(n, slot_off, o_off, i_off, sem):
        start, ln, a0 = node_params(n)
        d = start - a0
        len_c = jnp.minimum(ln, MAXL)
        add1 = jnp.where(ln <= MAXL, jnp.float32(1.0), jnp.float32(0.0))
        for t in range(38):
            w = lane + t * L
            p = w // 3
            gi = (w % 3) * 512 + p + d + slot_off
            x = plsc.load_gather(in_v, [gi])
            out_v[pl.ds(o_off + t * L, L)] = jnp.where(
                p < len_c, x + add1, jnp.float32(0.0))
        for t in range(13):
            p = lane + t * L
            x2 = in_v[pl.ds(slot_off + 1024 + d + t * L, L)]
            y = jnp.where(p < len_c, x2 + add1, jnp.float32(0.0))
            int_v[pl.ds(i_off + t * L, L)] = y.astype(jnp.int32)
        g = base + n
        pltpu.async_copy(out_v.at[pl.ds(o_off, WPN)],
                         pad_hbm.at[pl.ds(g * WPN, WPN)], sem)
        pltpu.async_copy(int_v.at[pl.ds(i_off, MAXL)],
                         int_hbm.at[pl.ds(g * MAXL, MAXL)], sem)

    issue_in(0, 0, sin0)

    def body(g, _):
        n0 = g * 2
        wait_in(0, sin0)
        issue_in(n0 + 1, 1536, sin1)

        @pl.when(g > 0)
        def _():
            wait_out(0, 0, sout0)

        compute(n0, 0, 0, 0, sout0)
        wait_in(1536, sin1)

        @pl.when(g < NPW // 2 - 1)
        def _():
            issue_in(n0 + 2, 0, sin0)

        @pl.when(g > 0)
        def _():
            wait_out(640, 256, sout1)

        compute(n0 + 1, 1536, 640, 256, sout1)
        return 0

    lax.fori_loop(0, NPW // 2, body, 0)
    wait_out(0, 0, sout0)
    wait_out(640, 256, sout1)

    for t in range(NPW // L):
        starts = cu_v[pl.ds(t * L, L)]
        ends = cu_v[pl.ds(t * L + 1, L)]
        len_v[pl.ds(t * L, L)] = jnp.minimum(ends - starts, MAXL)
    pltpu.sync_copy(len_v, len_hbm.at[pl.ds(base, NPW)])


_sharding = jax.sharding.SingleDeviceSharding(jax.devices()[0])
_out_formats = (
    Format(Layout((0, 1, 2), ((8, 3),)), _sharding),
    Format(Layout((0,), None), _sharding),
    Format(Layout((0, 1), ((8, 200),)), _sharding),
    Format(Layout((0, 1), ((8, 1),)), _sharding),
)


@jax.jit
def kernel(edges_flat, cu_seqlens, generated_data, target_quantile):
    e0 = edges_flat[:, 0]
    e1 = edges_flat[:, 1]
    e2 = edges_flat[:, 2]
    cu_pad = jnp.concatenate(
        [cu_seqlens, jnp.full((31,), TE, jnp.int32)])
    gen = generated_data.reshape(-1)

    comb = jnp.concatenate([gen, target_quantile])
    sorted_comb = _sort_kernel(comb)
    gidx = sorted_comb[:NN]
    st = lax.bitcast_convert_type(sorted_comb[NN:], jnp.float32)
    mapped = _final_kernel(gidx.reshape(NN // 128, 128), st)
    padded_flat, int_flat, lens = _pad_kernel(e0, e1, e2, cu_pad)

    return (padded_flat.reshape(NN, MAXL, 3), lens,
            int_flat.reshape(NN, MAXL), mapped.reshape(NN, 1))


# speedup vs baseline: 1.9268x; 1.0027x over previous
import functools

import jax
import jax.numpy as jnp
from jax import lax
from jax.experimental import pallas as pl
from jax.experimental.layout import Format, Layout, with_layout_constraint
from jax.experimental.pallas import tpu as pltpu
from jax.experimental.pallas import tpu_sc as plsc

NC, NS, L = 2, 16, 16
NW = NC * NS
MAXL = 200
NN = 16384
TE = 1638400
WPN = 3 * MAXL
NPW = NN // NW
SPS = NN // NS
NV = SPS // L

_mesh = plsc.VectorSubcoreMesh(
    core_axis_name="c", subcore_axis_name="s", num_cores=NC, num_subcores=NS
)
_params = pltpu.CompilerParams(needs_layout_passes=False,
                               use_tc_tiling_on_sc=False)

_I31 = 0x7FFFFFFF


def _lane():
    return lax.iota(jnp.int32, L)


@functools.partial(
    pl.kernel,
    out_type=jax.ShapeDtypeStruct((2 * NN,), jnp.int32),
    mesh=_mesh,
    compiler_params=_params,
    scratch_types=[
        pltpu.VMEM((SPS,), jnp.float32),
        pltpu.VMEM((SPS,), jnp.int32),
        pltpu.VMEM((SPS,), jnp.int32),
        pltpu.VMEM((SPS,), jnp.int32),
        pltpu.VMEM((SPS,), jnp.int32),
        pltpu.VMEM((SPS,), jnp.int32),
        pltpu.VMEM((SPS,), jnp.int32),
        pltpu.VMEM_SHARED((NN,), jnp.int32),
        pltpu.VMEM_SHARED((NN,), jnp.int32),
    ],
)
def _sort_kernel(comb_hbm, out_hbm,
                 fb, ka, va, kb, vb, pk, pv, shk, shv):
    c = lax.axis_index("c")
    s = lax.axis_index("s")
    base = s * SPS
    cbase = c * NN + base
    lane = _lane()

    pltpu.sync_copy(comb_hbm.at[pl.ds(cbase, SPS)], fb)

    for t in range(NV):
        i = plsc.bitcast(fb[pl.ds(t * L, L)], jnp.int32)
        ka[pl.ds(t * L, L)] = jnp.where(i >= 0, i, i ^ _I31)
        va[pl.ds(t * L, L)] = lane + (base + t * L)

    def do_pass(kk, jj, src_k, src_v, dst_k, dst_v):
        @pl.when(jj >= SPS)
        def _():
            pltpu.sync_copy(src_k, shk.at[pl.ds(base, SPS)])
            pltpu.sync_copy(src_v, shv.at[pl.ds(base, SPS)])
            plsc.subcore_barrier()
            ps = s ^ (jj // SPS)
            pltpu.sync_copy(shk.at[pl.ds(ps * SPS, SPS)], pk)
            pltpu.sync_copy(shv.at[pl.ds(ps * SPS, SPS)], pv)
            is_lo = (s & (jj // SPS)) == 0
            asc = (base & kk) == 0
            take_min = is_lo == asc

            def t_body(t, _):
                sl = pl.ds(t * L, L)
                xk, xv = src_k[sl], src_v[sl]
                yk, yv = pk[sl], pv[sl]
                ltv = (xk < yk) | ((xk == yk) & (xv < yv))
                cond = ltv == take_min
                dst_k[sl] = jnp.where(cond, xk, yk)
                dst_v[sl] = jnp.where(cond, xv, yv)
                return 0

            lax.fori_loop(0, NV, t_body, 0)
            plsc.subcore_barrier()

        @pl.when((jj >= L) & (jj < SPS))
        def _():
            jv = jj // L

            def q_body(q, _):
                lo = q & (jv - 1)
                t = (q - lo) * 2 + lo
                t2 = t + jv
                s1 = pl.ds(t * L, L)
                s2 = pl.ds(t2 * L, L)
                asc = ((base + t * L) & kk) == 0
                xk, xv = src_k[s1], src_v[s1]
                yk, yv = src_k[s2], src_v[s2]
                ltv = (xk < yk) | ((xk == yk) & (xv < yv))
                cond = ltv == asc
                dst_k[s1] = jnp.where(cond, xk, yk)
                dst_v[s1] = jnp.where(cond, xv, yv)
                dst_k[s2] = jnp.where(cond, yk, xk)
                dst_v[s2] = jnp.where(cond, yv, xv)
                return 0

            lax.fori_loop(0, NV // 2, q_body, 0)

        @pl.when(jj < L)
        def _():
            perm = lane ^ jj
            is_lo = (lane & jj) == 0

            def t_body(t, _):
                sl = pl.ds(t * L, L)
                xk, xv = src_k[sl], src_v[sl]
                gi = perm + t * L
                yk = plsc.load_gather(src_k, [gi])
                yv = plsc.load_gather(src_v, [gi])
                asc = (((base + t * L) + lane) & kk) == 0
                take_min = is_lo == asc
                ltv = (xk < yk) | ((xk == yk) & (xv < yv))
                cond = ltv == take_min
                dst_k[sl] = jnp.where(cond, xk, yk)
                dst_v[sl] = jnp.where(cond, xv, yv)
                return 0

            lax.fori_loop(0, NV, t_body, 0)

    def stage_body(st_i, ph):
        kk = lax.shift_left(jnp.int32(1), st_i)

        def j_body(m, ph):
            jj = lax.shift_left(jnp.int32(1), st_i - 1 - m)

            @pl.when(ph == 0)
            def _():
                do_pass(kk, jj, ka, va, kb, vb)

            @pl.when(ph == 1)
            def _():
                do_pass(kk, jj, kb, vb, ka, va)

            return ph ^ 1

        return lax.fori_loop(0, st_i, j_body, ph)

    lax.fori_loop(1, 15, stage_body, jnp.int32(0))

    for t in range(NV):
        sl = pl.ds(t * L, L)
        k = kb[sl]
        bits = jnp.where(k >= 0, k, k ^ _I31)
        pk[sl] = jnp.where(c == 0, vb[sl], bits)
    pltpu.sync_copy(pk, out_hbm.at[pl.ds(cbase, SPS)])


@functools.partial(
    pl.kernel,
    out_type=jax.ShapeDtypeStruct((NN,), jnp.int32),
    mesh=_mesh,
    compiler_params=_params,
    scratch_types=[
        pltpu.VMEM((NN,), jnp.float32),
        pltpu.VMEM((NPW // 128, 128), jnp.int32),
        pltpu.VMEM((NPW,), jnp.int32),
        pltpu.SemaphoreType.DMA,
    ],
)
def _final_kernel(gidx_hbm, st_hbm, map_hbm, st_v, gi_v, val_v, sem):
    c = lax.axis_index("c")
    s = lax.axis_index("s")
    wid = s * NC + c
    base = wid * NPW
    lane = _lane()

    pltpu.sync_copy(st_hbm, st_v)
    pltpu.sync_copy(gidx_hbm.at[pl.ds(wid * (NPW // 128), NPW // 128)], gi_v)

    nm1 = jnp.float32(NN - 1)
    for t in range(NPW // L):
        r = (base + t * L) + lane
        ii = (r.astype(jnp.float32) / nm1) * nm1
        fl = ii.astype(jnp.int32)
        flf = fl.astype(jnp.float32)
        ce = jnp.minimum(fl + (ii > flf).astype(jnp.int32), NN - 1)
        wc = ii - flf
        a = plsc.load_gather(st_v, [fl])
        b = plsc.load_gather(st_v, [ce])
        val = (jnp.float32(1.0) - wc) * a + wc * b
        val_v[pl.ds(t * L, L)] = val.astype(jnp.int32)

    for q in range(NPW // 128):
        pltpu.async_copy(
            val_v.at[pl.ds(q * 128, 128)], map_hbm.at[gi_v.at[q]], sem
        ).wait()


@functools.partial(
    pl.kernel,
    out_type=(
        jax.ShapeDtypeStruct((NN * WPN,), jnp.float32),
        jax.ShapeDtypeStruct((NN * MAXL,), jnp.int32),
        jax.ShapeDtypeStruct((NN,), jnp.int32),
    ),
    mesh=_mesh,
    compiler_params=_params,
    scratch_types=[
        pltpu.VMEM((NPW + 16,), jnp.int32),
        pltpu.VMEM((3072,), jnp.float32),
        pltpu.VMEM((1280,), jnp.float32),
        pltpu.VMEM((512,), jnp.int32),
        pltpu.VMEM((NPW,), jnp.int32),
        pltpu.SemaphoreType.DMA,
        pltpu.SemaphoreType.DMA,
        pltpu.SemaphoreType.DMA,
        pltpu.SemaphoreType.DMA,
    ],
)
def _pad_kernel(e0_hbm, e1_hbm, e2_hbm, cu_hbm, pad_hbm, int_hbm, len_hbm,
                cu_v, in_v, out_v, int_v, len_v,
                sin0, sin1, sout0, sout1):
    c = lax.axis_index("c")
    s = lax.axis_index("s")
    wid = s * NC + c
    base = wid * NPW
    lane = _lane()

    pltpu.sync_copy(cu_hbm.at[pl.ds(base, NPW + 16)], cu_v)

    def node_params(n):
        cuv = cu_v[pl.ds(n, L)]
        start = cuv[0]
        end = cuv[1]
        ln = end - start
        a0 = jnp.minimum(start - (start & 7), TE - 208)
        a0 = pl.multiple_of(a0, 8)
        return start, ln, a0

    def issue_in(n, slot_off, sem):
        _, _, a0 = node_params(n)
        pltpu.async_copy(e0_hbm.at[pl.ds(a0, 208)],
                         in_v.at[pl.ds(slot_off, 208)], sem)
        pltpu.async_copy(e1_hbm.at[pl.ds(a0, 208)],
                         in_v.at[pl.ds(slot_off + 512, 208)], sem)
        pltpu.async_copy(e2_hbm.at[pl.ds(a0, 208)],
                         in_v.at[pl.ds(slot_off + 1024, 208)], sem)

    def wait_in(slot_off, sem):
        pltpu.make_async_copy(e0_hbm.at[pl.ds(0, 208)],
                              in_v.at[pl.ds(slot_off, 208)], sem).wait()
        pltpu.make_async_copy(e1_hbm.at[pl.ds(0, 208)],
                              in_v.at[pl.ds(slot_off + 512, 208)], sem).wait()
        pltpu.make_async_copy(e2_hbm.at[pl.ds(0, 208)],
                              in_v.at[pl.ds(slot_off + 1024, 208)], sem).wait()

    def wait_out(o_off, i_off, sem):
        pltpu.make_async_copy(out_v.at[pl.ds(o_off, WPN)],
                              pad_hbm.at[pl.ds(0, WPN)], sem).wait()
        pltpu.make_async_copy(int_v.at[pl.ds(i_off, MAXL)],
                              int_hbm.at[pl.ds(0, MAXL)], sem).wait()

    def compute(n, slot_off, o_off, i_off, sem):
        start, ln, a0 = node_params(n)
        d = start - a0
        len_c = jnp.minimum(ln, MAXL)
        add1 = jnp.where(ln <= MAXL, jnp.float32(1.0), jnp.float32(0.0))
        for t in range(38):
            w = lane + t * L
            p = w // 3
            gi = (w % 3) * 512 + p + d + slot_off
            x = plsc.load_gather(in_v, [gi])
            out_v[pl.ds(o_off + t * L, L)] = jnp.where(
                p < len_c, x + add1, jnp.float32(0.0))
        for t in range(13):
            p = lane + t * L
            x2 = in_v[pl.ds(slot_off + 1024 + d + t * L, L)]
            y = jnp.where(p < len_c, x2 + add1, jnp.float32(0.0))
            int_v[pl.ds(i_off + t * L, L)] = y.astype(jnp.int32)
        g = base + n
        pltpu.async_copy(out_v.at[pl.ds(o_off, WPN)],
                         pad_hbm.at[pl.ds(g * WPN, WPN)], sem)
        pltpu.async_copy(int_v.at[pl.ds(i_off, MAXL)],
                         int_hbm.at[pl.ds(g * MAXL, MAXL)], sem)

    issue_in(0, 0, sin0)

    def body(g, _):
        n0 = g * 2
        wait_in(0, sin0)
        issue_in(n0 + 1, 1536, sin1)

        @pl.when(g > 0)
        def _():
            wait_out(0, 0, sout0)

        compute(n0, 0, 0, 0, sout0)
        wait_in(1536, sin1)

        @pl.when(g < NPW // 2 - 1)
        def _():
            issue_in(n0 + 2, 0, sin0)

        @pl.when(g > 0)
        def _():
            wait_out(640, 256, sout1)

        compute(n0 + 1, 1536, 640, 256, sout1)
        return 0

    lax.fori_loop(0, NPW // 2, body, 0)
    wait_out(0, 0, sout0)
    wait_out(640, 256, sout1)

    for t in range(NPW // L):
        starts = cu_v[pl.ds(t * L, L)]
        ends = cu_v[pl.ds(t * L + 1, L)]
        len_v[pl.ds(t * L, L)] = jnp.minimum(ends - starts, MAXL)
    pltpu.sync_copy(len_v, len_hbm.at[pl.ds(base, NPW)])


_sharding = jax.sharding.SingleDeviceSharding(jax.devices()[0])
_fmt_padded = Format(Layout((0, 1, 2), ((8, 3),)), _sharding)
_fmt_int = Format(Layout((0, 1), ((8, 200),)), _sharding)
_fmt_mapped = Format(Layout((0, 1), ((8, 1),)), _sharding)
_fmt_lens = Format(Layout((0,), None), _sharding)
_out_formats = (_fmt_padded, _fmt_lens, _fmt_int, _fmt_mapped)


@functools.partial(jax.jit, out_shardings=_out_formats)
def kernel(edges_flat, cu_seqlens, generated_data, target_quantile):
    e0 = edges_flat[:, 0]
    e1 = edges_flat[:, 1]
    e2 = edges_flat[:, 2]
    cu_pad = jnp.concatenate(
        [cu_seqlens, jnp.full((31,), TE, jnp.int32)])
    gen = generated_data.reshape(-1)

    comb = jnp.concatenate([gen, target_quantile])
    sorted_comb = _sort_kernel(comb)
    gidx = sorted_comb[:NN]
    st = lax.bitcast_convert_type(sorted_comb[NN:], jnp.float32)
    mapped = _final_kernel(gidx.reshape(NN // 128, 128), st)
    padded_flat, int_flat, lens = _pad_kernel(e0, e1, e2, cu_pad)

    padded = with_layout_constraint(
        padded_flat.reshape(NN, MAXL, 3), _fmt_padded.layout)
    int_tensor = with_layout_constraint(
        int_flat.reshape(NN, MAXL), _fmt_int.layout)
    mapped2 = with_layout_constraint(mapped.reshape(NN, 1), _fmt_mapped.layout)
    return padded, lens, int_tensor, mapped2
